# baseline (speedup 1.0000x reference)
import jax
import jax.numpy as jnp
from jax.experimental import pallas as pl
from jax.experimental.pallas import tpu as pltpu

_INV_LN2 = 1.4426950408889634
_LN2 = 0.6931471805599453
_ROWS = 8192
_COLS = 4096
_BLOCK_ROWS = 64
_GRID = _ROWS // _BLOCK_ROWS


def _jsd_block_kernel(p_ref, w_ref, out_ref):
    p = p_ref[...]
    w = w_ref[...]
    s = w + p
    t = jnp.where(w > 0, w * jnp.log(w), 0.0)
    t = t + jnp.where(p > 0, p * jnp.log(p), 0.0)
    t = t + jnp.where(s > 0, s * (_LN2 - jnp.log(s)), 0.0)
    out_ref[0] = jnp.sum(t, keepdims=True)


def kernel(P, W):
    partials = pl.pallas_call(
        _jsd_block_kernel,
        grid=(_GRID,),
        in_specs=[
            pl.BlockSpec((_BLOCK_ROWS, _COLS), lambda i: (i, 0)),
            pl.BlockSpec((_BLOCK_ROWS, _COLS), lambda i: (i, 0)),
        ],
        out_specs=pl.BlockSpec((1, 1, 1), lambda i: (i, 0, 0)),
        out_shape=jax.ShapeDtypeStruct((_GRID, 1, 1), jnp.float32),
        compiler_params=pltpu.CompilerParams(
            dimension_semantics=("parallel",)
        ),
    )(P, W)
    return jnp.sum(partials) * (0.5 * _INV_LN2 / _ROWS)


# speedup vs baseline: 1.3116x; 1.3116x over previous
import jax
import jax.numpy as jnp
from jax.experimental import pallas as pl
from jax.experimental.pallas import tpu as pltpu

_INV_LN2 = 1.4426950408889634
_LN2 = 0.6931471805599453
_ROWS = 8192
_COLS = 4096
_BLOCK_ROWS = 256
_CHUNK_ROWS = 8
_GRID = _ROWS // _BLOCK_ROWS


def _jsd_block_kernel(p_ref, w_ref, out_ref):
    acc = jnp.zeros((_CHUNK_ROWS, 128), jnp.float32)
    for r in range(0, _BLOCK_ROWS, _CHUNK_ROWS):
        p = p_ref[r:r + _CHUNK_ROWS, :]
        w = w_ref[r:r + _CHUNK_ROWS, :]
        s = w + p
        t = jnp.where(w > 0, w * jnp.log(w), 0.0)
        t = t + jnp.where(p > 0, p * jnp.log(p), 0.0)
        t = t + jnp.where(s > 0, s * (_LN2 - jnp.log(s)), 0.0)
        acc = acc + t.reshape(_CHUNK_ROWS, _COLS // 128, 128).sum(axis=1)
    out_ref[0] = jnp.sum(acc, keepdims=True)


def kernel(P, W):
    partials = pl.pallas_call(
        _jsd_block_kernel,
        grid=(_GRID,),
        in_specs=[
            pl.BlockSpec((_BLOCK_ROWS, _COLS), lambda i: (i, 0)),
            pl.BlockSpec((_BLOCK_ROWS, _COLS), lambda i: (i, 0)),
        ],
        out_specs=pl.BlockSpec((1, 1, 1), lambda i: (i, 0, 0)),
        out_shape=jax.ShapeDtypeStruct((_GRID, 1, 1), jnp.float32),
        compiler_params=pltpu.CompilerParams(
            dimension_semantics=("parallel",)
        ),
    )(P, W)
    return jnp.sum(partials) * (0.5 * _INV_LN2 / _ROWS)


# speedup vs baseline: 1.5809x; 1.2053x over previous
import jax
import jax.numpy as jnp
from jax.experimental import pallas as pl
from jax.experimental.pallas import tpu as pltpu

_TINY = 1e-30
_LN2 = 0.6931471805599453
_INV_LN2 = 1.4426950408889634
_ROWS = 8192
_COLS = 4096
_BLOCK_ROWS = 256
_CHUNK_ROWS = 8
_GRID = _ROWS // _BLOCK_ROWS


def _jsd_block_kernel(p_ref, w_ref, out_ref):
    q = _COLS // 4
    acc = jnp.zeros((_CHUNK_ROWS, q), jnp.float32)
    for r in range(0, _BLOCK_ROWS, _CHUNK_ROWS):
        for c in range(0, _COLS, q):
            p = p_ref[r:r + _CHUNK_ROWS, c:c + q]
            w = w_ref[r:r + _CHUNK_ROWS, c:c + q]
            s = w + p
            t = w * jnp.log(jnp.maximum(w, _TINY))
            t = t + p * jnp.log(jnp.maximum(p, _TINY))
            t = t + s * (_LN2 - jnp.log(jnp.maximum(s, _TINY)))
            acc = acc + t
    out_ref[0] = jnp.sum(acc, keepdims=True)


def kernel(P, W):
    partials = pl.pallas_call(
        _jsd_block_kernel,
        grid=(_GRID,),
        in_specs=[
            pl.BlockSpec((_BLOCK_ROWS, _COLS), lambda i: (i, 0)),
            pl.BlockSpec((_BLOCK_ROWS, _COLS), lambda i: (i, 0)),
        ],
        out_specs=pl.BlockSpec((1, 1, 1), lambda i: (i, 0, 0)),
        out_shape=jax.ShapeDtypeStruct((_GRID, 1, 1), jnp.float32),
        compiler_params=pltpu.CompilerParams(
            dimension_semantics=("parallel",)
        ),
    )(P, W)
    return jnp.sum(partials) * (0.5 * _INV_LN2 / _ROWS)
